# baseline (speedup 1.0000x reference)
import jax
import jax.numpy as jnp
from jax.experimental import pallas as pl
from jax.experimental.pallas import tpu as pltpu

N = 512
D = 128
BI = 32


def _prep_kernel(x_ref, ph_ref, w1r_ref, w1s_ref, b1_ref,
                 hr_ref, hs_ref, cs_ref):
    x = x_ref[...]
    hr_ref[...] = (jnp.dot(x, w1r_ref[...], preferred_element_type=jnp.float32)
                   + b1_ref[...]).astype(jnp.bfloat16)
    hs_ref[...] = jnp.dot(x, w1s_ref[...], preferred_element_type=jnp.float32
                          ).astype(jnp.bfloat16)
    ph = ph_ref[...]
    cs_ref[...] = jnp.concatenate(
        [jnp.cos(ph), jnp.sin(ph), jnp.ones_like(ph[:, :1])],
        axis=-1).astype(jnp.bfloat16)


def _mp_kernel(x_ref, adj_ref, hr_ref, hs_ref, cs_ref, w2_ref, b2_ref,
               wg2_ref, u1x_ref, u1a_ref, c1_ref, u2_ref, c2_ref, out_ref):
    i = pl.program_id(0)

    cs = cs_ref[...]
    csb = cs_ref[pl.ds(i * BI, BI), :]
    cd = (csb[:, None, :] * cs[None, :, :]).reshape(BI * N, cs.shape[-1])
    t = jnp.tanh(jnp.dot(cd, wg2_ref[...], preferred_element_type=jnp.float32
                         ).astype(jnp.bfloat16))

    hrb = hr_ref[pl.ds(i * BI, BI), :]
    hid = jax.nn.relu(
        (hrb[:, None, :] + hs_ref[...][None, :, :]).reshape(BI * N, D))
    msg = jnp.dot(hid, w2_ref[...], preferred_element_type=jnp.float32
                  ).astype(jnp.bfloat16)
    mt = msg * t

    m = (adj_ref[...] != 0).astype(jnp.bfloat16)
    counts = jnp.sum(m.astype(jnp.float32), axis=1, keepdims=True)
    bdot = lambda v: jax.lax.dot_general(
        m, v.reshape(BI, N, D),
        dimension_numbers=(((1,), (1,)), ((0,), (0,))),
        preferred_element_type=jnp.float32)
    total = bdot(msg) + bdot(mt) + b2_ref[...] * (counts + bdot(t))
    agg = total * (0.5 / jnp.maximum(counts, 1.0))

    xb = x_ref[pl.ds(i * BI, BI), :]
    h = jax.nn.relu(
        jnp.dot(xb, u1x_ref[...], preferred_element_type=jnp.float32)
        + jnp.dot(agg, u1a_ref[...], preferred_element_type=jnp.float32)
        + c1_ref[...])
    out_ref[...] = xb + jnp.dot(h, u2_ref[...],
                                preferred_element_type=jnp.float32) + c2_ref[...]


@jax.jit
def kernel(node_features, adjacency, node_phases, W1, b1, W2, b2, Wg, bg,
           U1, c1, U2, c2):
    d = node_features.shape[1]
    osc = node_phases.shape[1]
    full = lambda shape: pl.BlockSpec(shape, lambda i: (0,) * len(shape))
    wg2 = (jnp.concatenate([Wg, Wg, bg[None, :]], axis=0) * 0.5
           ).astype(jnp.bfloat16)

    hr, hs, cst = pl.pallas_call(
        _prep_kernel,
        out_shape=(
            jax.ShapeDtypeStruct((N, D), jnp.bfloat16),
            jax.ShapeDtypeStruct((N, D), jnp.bfloat16),
            jax.ShapeDtypeStruct((N, 2 * osc + 1), jnp.bfloat16),
        ),
    )(node_features, node_phases, W1[:d], W1[d:], b1)

    return pl.pallas_call(
        _mp_kernel,
        grid=(N // BI,),
        in_specs=[
            full((N, D)),
            pl.BlockSpec((BI, N), lambda i: (i, 0)),
            full((N, D)), full((N, D)),
            full((N, 2 * osc + 1)),
            full((D, D)), full((D,)),
            full((2 * osc + 1, D)),
            full((D, D)), full((D, D)), full((D,)),
            full((D, D)), full((D,)),
        ],
        out_specs=pl.BlockSpec((BI, D), lambda i: (i, 0)),
        out_shape=jax.ShapeDtypeStruct((N, D), jnp.float32),
    )(node_features, adjacency, hr, hs, cst,
      W2.astype(jnp.bfloat16), b2, wg2, U1[:d], U1[d:], c1, U2, c2)


# speedup vs baseline: 1.3946x; 1.3946x over previous
import jax
import jax.numpy as jnp
from jax.experimental import pallas as pl
from jax.experimental.pallas import tpu as pltpu

N = 512
D = 128
BI = 32


def _prep_kernel(x_ref, ph_ref, w1r_ref, w1s_ref, b1_ref,
                 hr_ref, hs_ref, cs_ref):
    x = x_ref[...]
    hr_ref[...] = (jnp.dot(x, w1r_ref[...], preferred_element_type=jnp.float32)
                   + b1_ref[...]).astype(jnp.bfloat16)
    hs_ref[...] = jnp.dot(x, w1s_ref[...], preferred_element_type=jnp.float32
                          ).astype(jnp.bfloat16)
    ph = ph_ref[...]
    cs_ref[...] = jnp.concatenate(
        [jnp.cos(ph), jnp.sin(ph), jnp.ones_like(ph[:, :1])],
        axis=-1).astype(jnp.bfloat16)


def _mp_kernel(x_ref, adj_ref, hr_ref, hs_ref, cs_ref, w2_ref, b2_ref,
               wg2_ref, u1x_ref, u1a_ref, c1_ref, u2_ref, c2_ref, out_ref):
    i = pl.program_id(0)

    cs = cs_ref[...]
    csb = cs_ref[pl.ds(i * BI, BI), :]
    cd = (csb[:, None, :] * cs[None, :, :]).reshape(BI * N, cs.shape[-1])
    t = jnp.tanh(jnp.dot(cd, wg2_ref[...],
                         preferred_element_type=jnp.float32))

    hrb = hr_ref[pl.ds(i * BI, BI), :]
    hid = jax.nn.relu(
        (hrb[:, None, :] + hs_ref[...][None, :, :]).reshape(BI * N, D))
    mb = jnp.dot(hid, w2_ref[...],
                 preferred_element_type=jnp.float32) + b2_ref[...]
    prod = mb * t + mb

    m = (adj_ref[...] != 0).astype(jnp.float32)
    counts = jnp.sum(m, axis=1, keepdims=True)
    msum = jax.lax.dot_general(
        m, prod.reshape(BI, N, D),
        dimension_numbers=(((1,), (1,)), ((0,), (0,))),
        preferred_element_type=jnp.float32)
    agg = msum * (0.5 / jnp.maximum(counts, 1.0))

    xb = x_ref[pl.ds(i * BI, BI), :]
    h = jax.nn.relu(
        jnp.dot(xb, u1x_ref[...], preferred_element_type=jnp.float32)
        + jnp.dot(agg, u1a_ref[...], preferred_element_type=jnp.float32)
        + c1_ref[...])
    out_ref[...] = xb + jnp.dot(h, u2_ref[...],
                                preferred_element_type=jnp.float32) + c2_ref[...]


@jax.jit
def kernel(node_features, adjacency, node_phases, W1, b1, W2, b2, Wg, bg,
           U1, c1, U2, c2):
    d = node_features.shape[1]
    osc = node_phases.shape[1]
    full = lambda shape: pl.BlockSpec(shape, lambda i: (0,) * len(shape))
    wg2 = (jnp.concatenate([Wg, Wg, bg[None, :]], axis=0) * 0.5
           ).astype(jnp.bfloat16)

    hr, hs, cst = pl.pallas_call(
        _prep_kernel,
        out_shape=(
            jax.ShapeDtypeStruct((N, D), jnp.bfloat16),
            jax.ShapeDtypeStruct((N, D), jnp.bfloat16),
            jax.ShapeDtypeStruct((N, 2 * osc + 1), jnp.bfloat16),
        ),
    )(node_features, node_phases, W1[:d], W1[d:], b1)

    return pl.pallas_call(
        _mp_kernel,
        grid=(N // BI,),
        in_specs=[
            full((N, D)),
            pl.BlockSpec((BI, N), lambda i: (i, 0)),
            full((N, D)), full((N, D)),
            full((N, 2 * osc + 1)),
            full((D, D)), full((D,)),
            full((2 * osc + 1, D)),
            full((D, D)), full((D, D)), full((D,)),
            full((D, D)), full((D,)),
        ],
        out_specs=pl.BlockSpec((BI, D), lambda i: (i, 0)),
        out_shape=jax.ShapeDtypeStruct((N, D), jnp.float32),
    )(node_features, adjacency, hr, hs, cst,
      W2.astype(jnp.bfloat16), b2, wg2, U1[:d], U1[d:], c1, U2, c2)


# speedup vs baseline: 1.4541x; 1.0427x over previous
import jax
import jax.numpy as jnp
from jax.experimental import pallas as pl
from jax.experimental.pallas import tpu as pltpu

N = 512
D = 128
BI = 32


def _mp_kernel(x_ref, adj_ref, ph_ref, w1r_ref, w1s_ref, b1_ref, w2_ref,
               b2_ref, wg2_ref, u1x_ref, u1a_ref, c1_ref, u2_ref, c2_ref,
               out_ref, hr_ref, hs_ref, cs_ref):
    i = pl.program_id(0)

    @pl.when(i == 0)
    def _prep():
        x = x_ref[...].astype(jnp.bfloat16)
        hr_ref[...] = (jnp.dot(x, w1r_ref[...],
                               preferred_element_type=jnp.float32)
                       + b1_ref[...]).astype(jnp.bfloat16)
        hs_ref[...] = jnp.dot(x, w1s_ref[...],
                              preferred_element_type=jnp.float32
                              ).astype(jnp.bfloat16)
        p = ph_ref[...]
        pr = jnp.where(p > jnp.float32(jnp.pi),
                       p - jnp.float32(2.0 * jnp.pi), p)
        z = pr * pr
        cs_ref[...] = (
            ((((jnp.float32(1.9064759e-05) * z + jnp.float32(-1.3440994e-03))
               * z + jnp.float32(4.1522268e-02)) * z
              + jnp.float32(-4.9983754e-01)) * z + jnp.float32(0.99997108))
        ).astype(jnp.bfloat16)

    cs = cs_ref[...]
    csb = cs_ref[pl.ds(i * BI, BI), :]
    cd = (csb[:, None, :] * cs[None, :, :]).reshape(BI * N, cs.shape[-1])
    t = jnp.tanh(jnp.dot(cd, wg2_ref[...],
                         preferred_element_type=jnp.float32))

    hrb = hr_ref[pl.ds(i * BI, BI), :]
    hid = jax.nn.relu(
        (hrb[:, None, :] + hs_ref[...][None, :, :]).reshape(BI * N, D))
    mb = jnp.dot(hid, w2_ref[...],
                 preferred_element_type=jnp.float32) + b2_ref[...]
    prod = mb * t + mb

    m = (adj_ref[...] != 0).astype(jnp.float32)
    counts = jnp.sum(m, axis=1, keepdims=True)
    msum = jax.lax.dot_general(
        m, prod.reshape(BI, N, D),
        dimension_numbers=(((1,), (1,)), ((0,), (0,))),
        preferred_element_type=jnp.float32)
    agg = msum * (0.5 / jnp.maximum(counts, 1.0))

    xb = x_ref[pl.ds(i * BI, BI), :]
    h = jax.nn.relu(
        jnp.dot(xb, u1x_ref[...], preferred_element_type=jnp.float32)
        + jnp.dot(agg, u1a_ref[...], preferred_element_type=jnp.float32)
        + c1_ref[...])
    out_ref[...] = xb + jnp.dot(h, u2_ref[...],
                                preferred_element_type=jnp.float32) + c2_ref[...]


@jax.jit
def kernel(node_features, adjacency, node_phases, W1, b1, W2, b2, Wg, bg,
           U1, c1, U2, c2):
    d = node_features.shape[1]
    osc = node_phases.shape[1]
    full = lambda shape: pl.BlockSpec(shape, lambda i: (0,) * len(shape))
    wg2 = (jnp.concatenate([Wg, Wg, bg[None, :]], axis=0) * 0.5
           ).astype(jnp.bfloat16)
    ph_aug = jnp.concatenate(
        [node_phases, node_phases - jnp.float32(jnp.pi / 2),
         jnp.zeros((N, 1), jnp.float32)], axis=1)
    return pl.pallas_call(
        _mp_kernel,
        grid=(N // BI,),
        in_specs=[
            full((N, D)),
            pl.BlockSpec((BI, N), lambda i: (i, 0)),
            full((N, 2 * osc + 1)),
            full((D, D)), full((D, D)), full((D,)),
            full((D, D)), full((D,)),
            full((2 * osc + 1, D)),
            full((D, D)), full((D, D)), full((D,)),
            full((D, D)), full((D,)),
        ],
        out_specs=pl.BlockSpec((BI, D), lambda i: (i, 0)),
        out_shape=jax.ShapeDtypeStruct((N, D), jnp.float32),
        scratch_shapes=[
            pltpu.VMEM((N, D), jnp.bfloat16),
            pltpu.VMEM((N, D), jnp.bfloat16),
            pltpu.VMEM((N, 2 * osc + 1), jnp.bfloat16),
        ],
    )(node_features, adjacency, ph_aug,
      W1[:d].astype(jnp.bfloat16), W1[d:].astype(jnp.bfloat16), b1,
      W2.astype(jnp.bfloat16), b2, wg2, U1[:d], U1[d:], c1, U2, c2)


# speedup vs baseline: 1.4997x; 1.0314x over previous
import jax
import jax.numpy as jnp
from jax.experimental import pallas as pl
from jax.experimental.pallas import tpu as pltpu

N = 512
D = 128
BI = 64


def _mp_kernel(x_ref, adj_ref, ph_ref, w1r_ref, w1s_ref, b1_ref, w2_ref,
               b2_ref, wg2_ref, u1x_ref, u1a_ref, c1_ref, u2_ref, c2_ref,
               out_ref, hr_ref, hs_ref, cs_ref):
    i = pl.program_id(0)

    @pl.when(i == 0)
    def _prep():
        x = x_ref[...].astype(jnp.bfloat16)
        hr_ref[...] = (jnp.dot(x, w1r_ref[...],
                               preferred_element_type=jnp.float32)
                       + b1_ref[...]).astype(jnp.bfloat16)
        hs_ref[...] = jnp.dot(x, w1s_ref[...],
                              preferred_element_type=jnp.float32
                              ).astype(jnp.bfloat16)
        p = ph_ref[...]
        pr = jnp.where(p > jnp.float32(jnp.pi),
                       p - jnp.float32(2.0 * jnp.pi), p)
        z = pr * pr
        cs_ref[...] = (
            ((((jnp.float32(1.9064759e-05) * z + jnp.float32(-1.3440994e-03))
               * z + jnp.float32(4.1522268e-02)) * z
              + jnp.float32(-4.9983754e-01)) * z + jnp.float32(0.99997108))
        ).astype(jnp.bfloat16)

    cs = cs_ref[...]
    csb = cs_ref[pl.ds(i * BI, BI), :]
    cd = (csb[:, None, :] * cs[None, :, :]).reshape(BI * N, cs.shape[-1])
    t = jnp.tanh(jnp.dot(cd, wg2_ref[...], preferred_element_type=jnp.float32
                         ).astype(jnp.bfloat16))

    hrb = hr_ref[pl.ds(i * BI, BI), :]
    hid = jax.nn.relu(
        (hrb[:, None, :] + hs_ref[...][None, :, :]).reshape(BI * N, D))
    mb = (jnp.dot(hid, w2_ref[...], preferred_element_type=jnp.float32
                  ).astype(jnp.bfloat16) + b2_ref[...])
    prod = mb * t + mb

    mf = (adj_ref[...] != 0).astype(jnp.float32)
    counts = jnp.sum(mf, axis=1, keepdims=True)
    msum = jax.lax.dot_general(
        mf.astype(jnp.bfloat16), prod.reshape(BI, N, D),
        dimension_numbers=(((1,), (1,)), ((0,), (0,))),
        preferred_element_type=jnp.float32)
    agg = msum * (0.5 / jnp.maximum(counts, 1.0))

    xb = x_ref[pl.ds(i * BI, BI), :]
    h = jax.nn.relu(
        jnp.dot(xb, u1x_ref[...], preferred_element_type=jnp.float32)
        + jnp.dot(agg, u1a_ref[...], preferred_element_type=jnp.float32)
        + c1_ref[...])
    out_ref[...] = xb + jnp.dot(h, u2_ref[...],
                                preferred_element_type=jnp.float32) + c2_ref[...]


@jax.jit
def kernel(node_features, adjacency, node_phases, W1, b1, W2, b2, Wg, bg,
           U1, c1, U2, c2):
    d = node_features.shape[1]
    osc = node_phases.shape[1]
    full = lambda shape: pl.BlockSpec(shape, lambda i: (0,) * len(shape))
    wg2 = (jnp.concatenate([Wg, Wg, bg[None, :]], axis=0) * 0.5
           ).astype(jnp.bfloat16)
    ph_aug = jnp.concatenate(
        [node_phases, node_phases - jnp.float32(jnp.pi / 2),
         jnp.zeros((N, 1), jnp.float32)], axis=1)
    return pl.pallas_call(
        _mp_kernel,
        grid=(N // BI,),
        in_specs=[
            full((N, D)),
            pl.BlockSpec((BI, N), lambda i: (i, 0)),
            full((N, 2 * osc + 1)),
            full((D, D)), full((D, D)), full((D,)),
            full((D, D)), full((D,)),
            full((2 * osc + 1, D)),
            full((D, D)), full((D, D)), full((D,)),
            full((D, D)), full((D,)),
        ],
        out_specs=pl.BlockSpec((BI, D), lambda i: (i, 0)),
        out_shape=jax.ShapeDtypeStruct((N, D), jnp.float32),
        scratch_shapes=[
            pltpu.VMEM((N, D), jnp.bfloat16),
            pltpu.VMEM((N, D), jnp.bfloat16),
            pltpu.VMEM((N, 2 * osc + 1), jnp.bfloat16),
        ],
    )(node_features, adjacency, ph_aug,
      W1[:d].astype(jnp.bfloat16), W1[d:].astype(jnp.bfloat16), b1,
      W2.astype(jnp.bfloat16), b2.astype(jnp.bfloat16), wg2,
      U1[:d], U1[d:], c1, U2, c2)


# speedup vs baseline: 1.6817x; 1.1213x over previous
import jax
import jax.numpy as jnp
from jax.experimental import pallas as pl
from jax.experimental.pallas import tpu as pltpu

N = 512
D = 128
BI = 64

_PI = 3.14159265358979
_TWO_PI = 6.28318530717959
_HALF_PI = 1.57079632679490


def _cos_poly(p):
    pr = jnp.where(p > _PI, p - _TWO_PI, p)
    z = pr * pr
    return ((((jnp.float32(1.9064759e-05) * z + jnp.float32(-1.3440994e-03))
              * z + jnp.float32(4.1522268e-02)) * z
             + jnp.float32(-4.9983754e-01)) * z + jnp.float32(0.99997108))


def _mp_kernel(x_ref, adj_ref, ph_ref, w1_ref, b1_ref, w2_ref, b2_ref,
               wg_ref, bg_ref, u1_ref, c1_ref, u2_ref, c2_ref, out_ref,
               hr_ref, hs_ref, cs_ref, w2b_ref, wg2_ref):
    i = pl.program_id(0)

    @pl.when(i == 0)
    def _prep():
        x = x_ref[...].astype(jnp.bfloat16)
        hr_ref[...] = (jnp.dot(x, w1_ref[:D].astype(jnp.bfloat16),
                               preferred_element_type=jnp.float32)
                       + b1_ref[...]).astype(jnp.bfloat16)
        hs_ref[...] = jnp.dot(x, w1_ref[D:].astype(jnp.bfloat16),
                              preferred_element_type=jnp.float32
                              ).astype(jnp.bfloat16)
        ph = ph_ref[...]
        cs_ref[...] = jnp.concatenate(
            [_cos_poly(ph), _cos_poly(ph - _HALF_PI),
             jnp.ones_like(ph[:, :1])], axis=-1).astype(jnp.bfloat16)
        w2b_ref[...] = w2_ref[...].astype(jnp.bfloat16)
        wg = wg_ref[...]
        wg2_ref[...] = (jnp.concatenate([wg, wg, bg_ref[...][None, :]], axis=0)
                        * jnp.float32(0.5)).astype(jnp.bfloat16)

    cs = cs_ref[...]
    csb = cs_ref[pl.ds(i * BI, BI), :]
    cd = (csb[:, None, :] * cs[None, :, :]).reshape(BI * N, cs.shape[-1])
    t = jnp.tanh(jnp.dot(cd, wg2_ref[...], preferred_element_type=jnp.float32
                         ).astype(jnp.bfloat16))

    hrb = hr_ref[pl.ds(i * BI, BI), :]
    hid = jax.nn.relu(
        (hrb[:, None, :] + hs_ref[...][None, :, :]).reshape(BI * N, D))
    mb = (jnp.dot(hid, w2b_ref[...], preferred_element_type=jnp.float32
                  ).astype(jnp.bfloat16) + b2_ref[...].astype(jnp.bfloat16))
    prod = mb * t + mb

    mf = (adj_ref[...] != 0).astype(jnp.float32)
    counts = jnp.sum(mf, axis=1, keepdims=True)
    msum = jax.lax.dot_general(
        mf.astype(jnp.bfloat16), prod.reshape(BI, N, D),
        dimension_numbers=(((1,), (1,)), ((0,), (0,))),
        preferred_element_type=jnp.float32)
    agg = msum * (0.5 / jnp.maximum(counts, 1.0))

    xb = x_ref[pl.ds(i * BI, BI), :]
    h = jax.nn.relu(
        jnp.dot(xb, u1_ref[:D], preferred_element_type=jnp.float32)
        + jnp.dot(agg, u1_ref[D:], preferred_element_type=jnp.float32)
        + c1_ref[...])
    out_ref[...] = xb + jnp.dot(h, u2_ref[...],
                                preferred_element_type=jnp.float32) + c2_ref[...]


@jax.jit
def kernel(node_features, adjacency, node_phases, W1, b1, W2, b2, Wg, bg,
           U1, c1, U2, c2):
    osc = node_phases.shape[1]
    full = lambda shape: pl.BlockSpec(shape, lambda i: (0,) * len(shape))
    return pl.pallas_call(
        _mp_kernel,
        grid=(N // BI,),
        in_specs=[
            full((N, D)),
            pl.BlockSpec((BI, N), lambda i: (i, 0)),
            full((N, osc)),
            full((2 * D, D)), full((D,)),
            full((D, D)), full((D,)),
            full((osc, D)), full((D,)),
            full((2 * D, D)), full((D,)),
            full((D, D)), full((D,)),
        ],
        out_specs=pl.BlockSpec((BI, D), lambda i: (i, 0)),
        out_shape=jax.ShapeDtypeStruct((N, D), jnp.float32),
        scratch_shapes=[
            pltpu.VMEM((N, D), jnp.bfloat16),
            pltpu.VMEM((N, D), jnp.bfloat16),
            pltpu.VMEM((N, 2 * osc + 1), jnp.bfloat16),
            pltpu.VMEM((D, D), jnp.bfloat16),
            pltpu.VMEM((2 * osc + 1, D), jnp.bfloat16),
        ],
    )(node_features, adjacency, node_phases, W1, b1, W2, b2, Wg, bg,
      U1, c1, U2, c2)
